# Initial kernel scaffold; baseline (speedup 1.0000x reference)
import functools

import jax
import jax.numpy as jnp
from jax import lax
from jax.experimental import pallas as pl
from jax.experimental.pallas import tpu as pltpu
from jax.experimental.pallas import tpu_sc as plsc

N = 10000
E = 320000
F = 128
NP = 10240
DW = 16
NC = 2
NS = 16
K = 80
ROWS_PER_TILE = NP // NS
EDGES_PER_CORE = E // NC
EDGES_PER_TILE = EDGES_PER_CORE // NS
CHUNKS = EDGES_PER_TILE // K


def _make_sc_agg():
    mesh = plsc.VectorSubcoreMesh(core_axis_name="c", subcore_axis_name="s")

    @functools.partial(
        pl.kernel,
        mesh=mesh,
        out_type=[
            jax.ShapeDtypeStruct((NC * NP, F), jnp.float32),
            jax.ShapeDtypeStruct((NC * NP, DW), jnp.float32),
        ],
        scratch_types=[
            pltpu.VMEM_SHARED((NP, F), jnp.float32),
            pltpu.VMEM_SHARED((NP, DW), jnp.float32),
            pltpu.VMEM((K,), jnp.int32),
            pltpu.VMEM((K,), jnp.int32),
            pltpu.VMEM((K, F), jnp.float32),
            pltpu.VMEM((K, DW), jnp.float32),
            pltpu.SemaphoreType.DMA,
        ],
    )
    def sc_agg(x_h, src_h, dst_h, ones_h, zrow_h, zdeg_h, s_out, d_out,
               s_sh, d_sh, src_v, dst_v, rows_v, ones_v, sem):
        c = lax.axis_index("c")
        s = lax.axis_index("s")
        rbase = s * ROWS_PER_TILE

        for j in range(ROWS_PER_TILE // 128):
            pltpu.sync_copy(zrow_h, s_sh.at[pl.ds(rbase + j * 128, 128)])
        pltpu.sync_copy(zdeg_h, d_sh.at[pl.ds(rbase, ROWS_PER_TILE)])
        pltpu.sync_copy(ones_h, ones_v)
        plsc.subcore_barrier()

        ebase = c * EDGES_PER_CORE + s * EDGES_PER_TILE

        def chunk(i, carry):
            off = pl.multiple_of(ebase + i * K, 8)
            pltpu.sync_copy(src_h.at[pl.ds(off, K)], src_v)
            pltpu.sync_copy(dst_h.at[pl.ds(off, K)], dst_v)
            pltpu.async_copy(x_h.at[src_v], rows_v, sem).wait()
            pltpu.sync_copy(rows_v, s_sh.at[dst_v], add=True)
            pltpu.sync_copy(ones_v, d_sh.at[dst_v], add=True)
            return carry

        lax.fori_loop(0, CHUNKS, chunk, 0)
        plsc.subcore_barrier()

        obase = c * NP + rbase
        pltpu.sync_copy(s_sh.at[pl.ds(rbase, ROWS_PER_TILE)],
                        s_out.at[pl.ds(obase, ROWS_PER_TILE)])
        pltpu.sync_copy(d_sh.at[pl.ds(rbase, ROWS_PER_TILE)],
                        d_out.at[pl.ds(obase, ROWS_PER_TILE)])

    return sc_agg


_sc_agg = _make_sc_agg()

BLK = 200


def _tc_body(x_ref, s0_ref, s1_ref, d0_ref, d1_ref, w_ref, b_ref, o_ref):
    xb = x_ref[...]
    agg = s0_ref[0] + s1_ref[0]
    deg = d0_ref[0][:, 0:1] + d1_ref[0][:, 0:1]
    agg = agg / jnp.maximum(deg, 1.0)
    h = (jnp.dot(xb, w_ref[:F, :], preferred_element_type=jnp.float32)
         + jnp.dot(agg, w_ref[F:, :], preferred_element_type=jnp.float32)
         + b_ref[...])
    o_ref[...] = jnp.maximum(h, 0.0)


def _tc_out(x, s_part, d_part, weight, bias):
    return pl.pallas_call(
        _tc_body,
        grid=(N // BLK,),
        in_specs=[
            pl.BlockSpec((BLK, F), lambda i: (i, 0)),
            pl.BlockSpec((1, BLK, F), lambda i: (0, i, 0)),
            pl.BlockSpec((1, BLK, F), lambda i: (1, i, 0)),
            pl.BlockSpec((1, BLK, DW), lambda i: (0, i, 0)),
            pl.BlockSpec((1, BLK, DW), lambda i: (1, i, 0)),
            pl.BlockSpec((2 * F, F), lambda i: (0, 0)),
            pl.BlockSpec((1, F), lambda i: (0, 0)),
        ],
        out_specs=pl.BlockSpec((BLK, F), lambda i: (i, 0)),
        out_shape=jax.ShapeDtypeStruct((N, F), jnp.float32),
    )(x, s_part, s_part, d_part, d_part, weight, bias.reshape(1, F))


def kernel(x, edge_index, weight, bias):
    dst = edge_index[0]
    src = edge_index[1]
    ones = jnp.ones((K, DW), jnp.float32)
    zrow = jnp.zeros((128, F), jnp.float32)
    zdeg = jnp.zeros((ROWS_PER_TILE, DW), jnp.float32)
    s_flat, d_flat = _sc_agg(x, src, dst, ones, zrow, zdeg)
    s_part = s_flat.reshape(NC, NP, F)
    d_part = d_flat.reshape(NC, NP, DW)
    return _tc_out(x, s_part, d_part, weight, bias)


# speedup vs baseline: 4.2427x; 4.2427x over previous
import functools

import jax
import jax.numpy as jnp
from jax import lax
from jax.experimental import pallas as pl
from jax.experimental.pallas import tpu as pltpu
from jax.experimental.pallas import tpu_sc as plsc

N = 10000
E = 320000
F = 128
NP = 10240
NC = 2
NS = 16
K = 80
ROWS_PER_TILE = NP // NS
EDGES_PER_CORE = E // NC
EDGES_PER_TILE = EDGES_PER_CORE // NS
CHUNKS = EDGES_PER_TILE // K
SLABS = ROWS_PER_TILE // K


def _make_sc_agg():
    mesh = plsc.VectorSubcoreMesh(core_axis_name="c", subcore_axis_name="s")

    @functools.partial(
        pl.kernel,
        mesh=mesh,
        out_type=[
            jax.ShapeDtypeStruct((NC * NP, F), jnp.float32),
            jax.ShapeDtypeStruct((NC * NP, F), jnp.float32),
        ],
        scratch_types=[
            pltpu.VMEM_SHARED((NP, F), jnp.float32),
            pltpu.VMEM((K,), jnp.int32),
            pltpu.VMEM((K,), jnp.int32),
            pltpu.VMEM((K, F), jnp.float32),
            pltpu.SemaphoreType.DMA,
        ],
    )
    def sc_agg(x_h, src_h, dst_h, zrow_h, ones_h, s_out, d_out,
               s_sh, src_v, dst_v, rows_v, sem):
        c = lax.axis_index("c")
        s = lax.axis_index("s")
        rbase = s * ROWS_PER_TILE
        obase = c * NP + rbase
        ebase = c * EDGES_PER_CORE + s * EDGES_PER_TILE

        def zero_own_slice():
            pltpu.sync_copy(zrow_h, rows_v)
            for j in range(SLABS):
                pltpu.sync_copy(rows_v, s_sh.at[pl.ds(rbase + j * K, K)])

        def writeback(dst_hbm):
            for j in range(SLABS):
                pltpu.sync_copy(s_sh.at[pl.ds(rbase + j * K, K)], rows_v)
                pltpu.sync_copy(rows_v, dst_hbm.at[pl.ds(obase + j * K, K)])

        zero_own_slice()
        plsc.subcore_barrier()

        def chunk_sum(i, carry):
            off = pl.multiple_of(ebase + i * K, 8)
            pltpu.sync_copy(src_h.at[pl.ds(off, K)], src_v)
            pltpu.sync_copy(dst_h.at[pl.ds(off, K)], dst_v)
            pltpu.async_copy(x_h.at[src_v], rows_v, sem).wait()
            pltpu.sync_copy(rows_v, s_sh.at[dst_v], add=True)
            return carry

        lax.fori_loop(0, CHUNKS, chunk_sum, 0)
        plsc.subcore_barrier()
        writeback(s_out)
        plsc.subcore_barrier()

        zero_own_slice()
        plsc.subcore_barrier()
        pltpu.sync_copy(ones_h, rows_v)

        def chunk_deg(i, carry):
            off = pl.multiple_of(ebase + i * K, 8)
            pltpu.sync_copy(dst_h.at[pl.ds(off, K)], dst_v)
            pltpu.sync_copy(rows_v, s_sh.at[dst_v], add=True)
            return carry

        lax.fori_loop(0, CHUNKS, chunk_deg, 0)
        plsc.subcore_barrier()
        writeback(d_out)

    return sc_agg


@functools.lru_cache(maxsize=1)
def _get_sc_agg():
    return _make_sc_agg()


BLK = 200


def _tc_body(x_ref, s0_ref, s1_ref, d0_ref, d1_ref, w_ref, b_ref, o_ref):
    xb = x_ref[...]
    agg = s0_ref[0] + s1_ref[0]
    deg = d0_ref[0][:, 0:1] + d1_ref[0][:, 0:1]
    agg = agg / jnp.maximum(deg, 1.0)
    h = (jnp.dot(xb, w_ref[:F, :], preferred_element_type=jnp.float32)
         + jnp.dot(agg, w_ref[F:, :], preferred_element_type=jnp.float32)
         + b_ref[...])
    o_ref[...] = jnp.maximum(h, 0.0)


def _tc_out(x, s_part, d_part, weight, bias):
    return pl.pallas_call(
        _tc_body,
        grid=(N // BLK,),
        in_specs=[
            pl.BlockSpec((BLK, F), lambda i: (i, 0)),
            pl.BlockSpec((1, BLK, F), lambda i: (0, i, 0)),
            pl.BlockSpec((1, BLK, F), lambda i: (1, i, 0)),
            pl.BlockSpec((1, BLK, F), lambda i: (0, i, 0)),
            pl.BlockSpec((1, BLK, F), lambda i: (1, i, 0)),
            pl.BlockSpec((2 * F, F), lambda i: (0, 0)),
            pl.BlockSpec((1, F), lambda i: (0, 0)),
        ],
        out_specs=pl.BlockSpec((BLK, F), lambda i: (i, 0)),
        out_shape=jax.ShapeDtypeStruct((N, F), jnp.float32),
    )(x, s_part, s_part, d_part, d_part, weight, bias.reshape(1, F))


def kernel(x, edge_index, weight, bias):
    dst = edge_index[0]
    src = edge_index[1]
    zrow = jnp.zeros((K, F), jnp.float32)
    ones = jnp.ones((K, F), jnp.float32)
    s_flat, d_flat = _get_sc_agg()(x, src, dst, zrow, ones)
    s_part = s_flat.reshape(NC, NP, F)
    d_part = d_flat.reshape(NC, NP, F)
    return _tc_out(x, s_part, d_part, weight, bias)


# speedup vs baseline: 6.5629x; 1.5469x over previous
import functools

import jax
import jax.numpy as jnp
from jax import lax
from jax.experimental import pallas as pl
from jax.experimental.pallas import tpu as pltpu
from jax.experimental.pallas import tpu_sc as plsc

N = 10000
E = 320000
F = 128
NP = 10240
NC = 2
NS = 16
K = 80
ROWS_PER_TILE = NP // NS
EDGES_PER_CORE = E // NC
EDGES_PER_TILE = EDGES_PER_CORE // NS
CHUNKS = EDGES_PER_TILE // K
SLABS = ROWS_PER_TILE // K


def _make_sc_agg():
    mesh = plsc.VectorSubcoreMesh(core_axis_name="c", subcore_axis_name="s")

    @functools.partial(
        pl.kernel,
        mesh=mesh,
        out_type=[
            jax.ShapeDtypeStruct((NC * NP, F), jnp.float32),
            jax.ShapeDtypeStruct((NC * NP, F), jnp.float32),
        ],
        scratch_types=[
            pltpu.VMEM_SHARED((NP, F), jnp.float32),
            pltpu.VMEM((K,), jnp.int32),
            pltpu.VMEM((K,), jnp.int32),
            pltpu.VMEM((K,), jnp.int32),
            pltpu.VMEM((K,), jnp.int32),
            pltpu.VMEM((K, F), jnp.float32),
            pltpu.VMEM((K, F), jnp.float32),
            pltpu.SemaphoreType.DMA,
            pltpu.SemaphoreType.DMA,
        ],
    )
    def sc_agg(x_h, src_h, dst_h, zrow_h, ones_h, s_out, d_out,
               s_sh, src_a, dst_a, src_b, dst_b, rows_a, rows_b,
               sem_a, sem_b):
        c = lax.axis_index("c")
        s = lax.axis_index("s")
        rbase = s * ROWS_PER_TILE
        obase = c * NP + rbase
        ebase = c * EDGES_PER_CORE + s * EDGES_PER_TILE

        def zero_own_slice():
            pltpu.sync_copy(zrow_h, rows_a)
            for j in range(SLABS):
                pltpu.sync_copy(rows_a, s_sh.at[pl.ds(rbase + j * K, K)])

        def writeback(dst_hbm):
            for j in range(SLABS):
                pltpu.sync_copy(s_sh.at[pl.ds(rbase + j * K, K)], rows_a)
                pltpu.sync_copy(rows_a, dst_hbm.at[pl.ds(obase + j * K, K)])

        def load_idx(i, src_v, dst_v):
            off = pl.multiple_of(ebase + i * K, 8)
            pltpu.sync_copy(src_h.at[pl.ds(off, K)], src_v)
            pltpu.sync_copy(dst_h.at[pl.ds(off, K)], dst_v)

        zero_own_slice()
        plsc.subcore_barrier()

        load_idx(0, src_a, dst_a)
        ga = pltpu.async_copy(x_h.at[src_a], rows_a, sem_a)
        load_idx(1, src_b, dst_b)
        gb = pltpu.async_copy(x_h.at[src_b], rows_b, sem_b)

        def pair_sum(p, carry):
            i0 = p * 2
            ga.wait()
            pltpu.sync_copy(rows_a, s_sh.at[dst_a], add=True)

            @pl.when(i0 + 2 < CHUNKS)
            def _():
                load_idx(i0 + 2, src_a, dst_a)
                pltpu.async_copy(x_h.at[src_a], rows_a, sem_a)

            gb.wait()
            pltpu.sync_copy(rows_b, s_sh.at[dst_b], add=True)

            @pl.when(i0 + 3 < CHUNKS)
            def _():
                load_idx(i0 + 3, src_b, dst_b)
                pltpu.async_copy(x_h.at[src_b], rows_b, sem_b)

            return carry

        lax.fori_loop(0, CHUNKS // 2, pair_sum, 0)
        ga.wait()
        pltpu.sync_copy(rows_a, s_sh.at[dst_a], add=True)

        plsc.subcore_barrier()
        writeback(s_out)
        plsc.subcore_barrier()

        zero_own_slice()
        plsc.subcore_barrier()
        pltpu.sync_copy(ones_h, rows_b)

        def load_dst(i, dst_v, sem):
            off = pl.multiple_of(ebase + i * K, 8)
            return pltpu.async_copy(dst_h.at[pl.ds(off, K)], dst_v, sem)

        da = load_dst(0, dst_a, sem_a)
        db = load_dst(1, dst_b, sem_b)

        def pair_deg(p, carry):
            i0 = p * 2
            da.wait()
            pltpu.sync_copy(rows_b, s_sh.at[dst_a], add=True)

            @pl.when(i0 + 2 < CHUNKS)
            def _():
                load_dst(i0 + 2, dst_a, sem_a)

            db.wait()
            pltpu.sync_copy(rows_b, s_sh.at[dst_b], add=True)

            @pl.when(i0 + 3 < CHUNKS)
            def _():
                load_dst(i0 + 3, dst_b, sem_b)

            return carry

        lax.fori_loop(0, CHUNKS // 2, pair_deg, 0)
        da.wait()
        pltpu.sync_copy(rows_b, s_sh.at[dst_a], add=True)

        plsc.subcore_barrier()
        writeback(d_out)

    return sc_agg


@functools.lru_cache(maxsize=1)
def _get_sc_agg():
    return _make_sc_agg()


BLK = 200


def _tc_body(x_ref, s0_ref, s1_ref, d0_ref, d1_ref, w_ref, b_ref, o_ref):
    xb = x_ref[...]
    agg = s0_ref[0] + s1_ref[0]
    deg = d0_ref[0][:, 0:1] + d1_ref[0][:, 0:1]
    agg = agg / jnp.maximum(deg, 1.0)
    h = (jnp.dot(xb, w_ref[:F, :], preferred_element_type=jnp.float32)
         + jnp.dot(agg, w_ref[F:, :], preferred_element_type=jnp.float32)
         + b_ref[...])
    o_ref[...] = jnp.maximum(h, 0.0)


def _tc_out(x, s_part, d_part, weight, bias):
    return pl.pallas_call(
        _tc_body,
        grid=(N // BLK,),
        in_specs=[
            pl.BlockSpec((BLK, F), lambda i: (i, 0)),
            pl.BlockSpec((1, BLK, F), lambda i: (0, i, 0)),
            pl.BlockSpec((1, BLK, F), lambda i: (1, i, 0)),
            pl.BlockSpec((1, BLK, F), lambda i: (0, i, 0)),
            pl.BlockSpec((1, BLK, F), lambda i: (1, i, 0)),
            pl.BlockSpec((2 * F, F), lambda i: (0, 0)),
            pl.BlockSpec((1, F), lambda i: (0, 0)),
        ],
        out_specs=pl.BlockSpec((BLK, F), lambda i: (i, 0)),
        out_shape=jax.ShapeDtypeStruct((N, F), jnp.float32),
    )(x, s_part, s_part, d_part, d_part, weight, bias.reshape(1, F))


def kernel(x, edge_index, weight, bias):
    dst = edge_index[0]
    src = edge_index[1]
    zrow = jnp.zeros((K, F), jnp.float32)
    ones = jnp.ones((K, F), jnp.float32)
    s_flat, d_flat = _get_sc_agg()(x, src, dst, zrow, ones)
    s_part = s_flat.reshape(NC, NP, F)
    d_part = d_flat.reshape(NC, NP, F)
    return _tc_out(x, s_part, d_part, weight, bias)


# speedup vs baseline: 7.3086x; 1.1136x over previous
import functools

import jax
import jax.numpy as jnp
from jax import lax
from jax.experimental import pallas as pl
from jax.experimental.pallas import tpu as pltpu
from jax.experimental.pallas import tpu_sc as plsc

N = 10000
E = 320000
F = 128
NP = 10240
NC = 2
NS = 16
K = 128
KT = 16
ROWS_PER_TILE = NP // NS
EDGES_PER_CORE = E // NC
EDGES_PER_TILE = EDGES_PER_CORE // NS
CHUNKS = EDGES_PER_TILE // K
SLABS = ROWS_PER_TILE // K


def _make_sc_agg():
    mesh = plsc.VectorSubcoreMesh(core_axis_name="c", subcore_axis_name="s")

    @functools.partial(
        pl.kernel,
        mesh=mesh,
        out_type=[
            jax.ShapeDtypeStruct((NC * NP, F), jnp.float32),
            jax.ShapeDtypeStruct((NC * NP, F), jnp.float32),
        ],
        scratch_types=[
            pltpu.VMEM_SHARED((NP, F), jnp.float32),
            pltpu.VMEM((K,), jnp.int32),
            pltpu.VMEM((K,), jnp.int32),
            pltpu.VMEM((K,), jnp.int32),
            pltpu.VMEM((K,), jnp.int32),
            pltpu.VMEM((K, F), jnp.float32),
            pltpu.VMEM((K, F), jnp.float32),
            pltpu.VMEM((KT,), jnp.int32),
            pltpu.VMEM((KT,), jnp.int32),
            pltpu.VMEM((KT, F), jnp.float32),
            pltpu.SemaphoreType.DMA,
            pltpu.SemaphoreType.DMA,
        ],
    )
    def sc_agg(x_h, src_h, dst_h, zrow_h, ones_h, s_out, d_out,
               s_sh, src_a, dst_a, src_b, dst_b, rows_a, rows_b,
               src_t, dst_t, rows_t, sem_a, sem_b):
        c = lax.axis_index("c")
        s = lax.axis_index("s")
        rbase = s * ROWS_PER_TILE
        obase = c * NP + rbase
        ebase = c * EDGES_PER_CORE + s * EDGES_PER_TILE

        def zero_own_slice():
            pltpu.sync_copy(zrow_h, rows_a)
            for j in range(SLABS):
                pltpu.sync_copy(rows_a, s_sh.at[pl.ds(rbase + j * K, K)])

        def writeback(dst_hbm):
            for j in range(SLABS):
                pltpu.sync_copy(s_sh.at[pl.ds(rbase + j * K, K)], rows_a)
                pltpu.sync_copy(rows_a, dst_hbm.at[pl.ds(obase + j * K, K)])

        def load_idx(i, src_v, dst_v):
            off = pl.multiple_of(ebase + i * K, 8)
            pltpu.sync_copy(src_h.at[pl.ds(off, K)], src_v)
            pltpu.sync_copy(dst_h.at[pl.ds(off, K)], dst_v)

        zero_own_slice()
        plsc.subcore_barrier()

        load_idx(0, src_a, dst_a)
        ga = pltpu.async_copy(x_h.at[src_a], rows_a, sem_a)
        load_idx(1, src_b, dst_b)
        gb = pltpu.async_copy(x_h.at[src_b], rows_b, sem_b)

        def pair_sum(p, carry):
            i0 = p * 2
            ga.wait()
            pltpu.sync_copy(rows_a, s_sh.at[dst_a], add=True)

            @pl.when(i0 + 2 < CHUNKS)
            def _():
                load_idx(i0 + 2, src_a, dst_a)
                pltpu.async_copy(x_h.at[src_a], rows_a, sem_a)

            gb.wait()
            pltpu.sync_copy(rows_b, s_sh.at[dst_b], add=True)

            @pl.when(i0 + 3 < CHUNKS)
            def _():
                load_idx(i0 + 3, src_b, dst_b)
                pltpu.async_copy(x_h.at[src_b], rows_b, sem_b)

            return carry

        lax.fori_loop(0, CHUNKS // 2, pair_sum, 0)
        toff = pl.multiple_of(ebase + CHUNKS * K, 8)
        pltpu.sync_copy(src_h.at[pl.ds(toff, KT)], src_t)
        pltpu.sync_copy(dst_h.at[pl.ds(toff, KT)], dst_t)
        pltpu.async_copy(x_h.at[src_t], rows_t, sem_a).wait()
        pltpu.sync_copy(rows_t, s_sh.at[dst_t], add=True)

        plsc.subcore_barrier()
        writeback(s_out)
        plsc.subcore_barrier()

        zero_own_slice()
        plsc.subcore_barrier()
        pltpu.sync_copy(ones_h, rows_b)

        def load_dst(i, dst_v, sem):
            off = pl.multiple_of(ebase + i * K, 8)
            return pltpu.async_copy(dst_h.at[pl.ds(off, K)], dst_v, sem)

        da = load_dst(0, dst_a, sem_a)
        db = load_dst(1, dst_b, sem_b)

        def pair_deg(p, carry):
            i0 = p * 2
            da.wait()
            pltpu.sync_copy(rows_b, s_sh.at[dst_a], add=True)

            @pl.when(i0 + 2 < CHUNKS)
            def _():
                load_dst(i0 + 2, dst_a, sem_a)

            db.wait()
            pltpu.sync_copy(rows_b, s_sh.at[dst_b], add=True)

            @pl.when(i0 + 3 < CHUNKS)
            def _():
                load_dst(i0 + 3, dst_b, sem_b)

            return carry

        lax.fori_loop(0, CHUNKS // 2, pair_deg, 0)
        pltpu.sync_copy(dst_h.at[pl.ds(toff, KT)], dst_t)
        pltpu.sync_copy(ones_h.at[pl.ds(0, KT)], rows_t)
        pltpu.sync_copy(rows_t, s_sh.at[dst_t], add=True)

        plsc.subcore_barrier()
        writeback(d_out)

    return sc_agg


@functools.lru_cache(maxsize=1)
def _get_sc_agg():
    return _make_sc_agg()


BLK = 200


def _tc_body(x_ref, s0_ref, s1_ref, d0_ref, d1_ref, w_ref, b_ref, o_ref):
    xb = x_ref[...]
    agg = s0_ref[0] + s1_ref[0]
    deg = d0_ref[0][:, 0:1] + d1_ref[0][:, 0:1]
    agg = agg / jnp.maximum(deg, 1.0)
    h = (jnp.dot(xb, w_ref[:F, :], preferred_element_type=jnp.float32)
         + jnp.dot(agg, w_ref[F:, :], preferred_element_type=jnp.float32)
         + b_ref[...])
    o_ref[...] = jnp.maximum(h, 0.0)


def _tc_out(x, s_part, d_part, weight, bias):
    return pl.pallas_call(
        _tc_body,
        grid=(N // BLK,),
        in_specs=[
            pl.BlockSpec((BLK, F), lambda i: (i, 0)),
            pl.BlockSpec((1, BLK, F), lambda i: (0, i, 0)),
            pl.BlockSpec((1, BLK, F), lambda i: (1, i, 0)),
            pl.BlockSpec((1, BLK, F), lambda i: (0, i, 0)),
            pl.BlockSpec((1, BLK, F), lambda i: (1, i, 0)),
            pl.BlockSpec((2 * F, F), lambda i: (0, 0)),
            pl.BlockSpec((1, F), lambda i: (0, 0)),
        ],
        out_specs=pl.BlockSpec((BLK, F), lambda i: (i, 0)),
        out_shape=jax.ShapeDtypeStruct((N, F), jnp.float32),
    )(x, s_part, s_part, d_part, d_part, weight, bias.reshape(1, F))


def kernel(x, edge_index, weight, bias):
    dst = edge_index[0]
    src = edge_index[1]
    zrow = jnp.zeros((K, F), jnp.float32)
    ones = jnp.ones((K, F), jnp.float32)
    s_flat, d_flat = _get_sc_agg()(x, src, dst, zrow, ones)
    s_part = s_flat.reshape(NC, NP, F)
    d_part = d_flat.reshape(NC, NP, F)
    return _tc_out(x, s_part, d_part, weight, bias)
